# Initial kernel scaffold; baseline (speedup 1.0000x reference)
import jax
import jax.numpy as jnp
from jax.experimental import pallas as pl


def kernel(x, token_table, pos_table):
    raise NotImplementedError("write your pallas kernel here")


# speedup vs baseline: 2.1722x; 2.1722x over previous
import functools

import jax
import jax.numpy as jnp
from jax import lax
from jax.experimental import pallas as pl
from jax.experimental.pallas import tpu as pltpu
from jax.experimental.pallas import tpu_sc as plsc

VOCAB = 100000
MAX_LEN = 200
EMBED_DIM = 64
BATCH = 1024
SEQ = 200

NUM_CORES = 2
NUM_SUBCORES = 16
NUM_WORKERS = NUM_CORES * NUM_SUBCORES
ROWS_PER_WORKER = BATCH * SEQ // NUM_WORKERS
CHUNK = 40
CHUNKS_PER_WORKER = ROWS_PER_WORKER // CHUNK
LANES = 16
DGROUPS = EMBED_DIM // LANES


def _sc_kernel_body(x_hbm, tok_hbm, pos_hbm, out_hbm,
                    idx_v, pos_v, buf0, buf1, sem0, sem1):
    wid = lax.axis_index("s") * NUM_CORES + lax.axis_index("c")
    base = wid * ROWS_PER_WORKER

    pltpu.sync_copy(x_hbm.at[pl.ds(base, ROWS_PER_WORKER)], idx_v)
    pltpu.sync_copy(pos_hbm, pos_v)

    bufs = (buf0, buf1)
    sems = (sem0, sem1)

    def start_gather(c, slot):
        idx = idx_v.at[pl.ds(c * CHUNK, CHUNK)]
        pltpu.make_async_copy(tok_hbm.at[idx], bufs[slot], sems[slot]).start()

    def wait_gather(slot):
        pltpu.make_async_copy(tok_hbm.at[idx_v.at[pl.ds(0, CHUNK)]],
                              bufs[slot], sems[slot]).wait()

    def add_pos_and_emit(c, slot):
        buf = bufs[slot]
        s0 = lax.rem(c, SEQ // CHUNK) * CHUNK

        def body(r, carry):
            for j in range(DGROUPS):
                sl = pl.ds(j * LANES, LANES)
                buf[r, sl] = buf[r, sl] + pos_v[s0 + r, sl]
            return carry

        lax.fori_loop(0, CHUNK, body, 0)
        pltpu.sync_copy(buf, out_hbm.at[pl.ds(base + c * CHUNK, CHUNK)])

    start_gather(0, 0)

    def outer(g, carry):
        for b in range(2):
            c = g * 2 + b
            slot = b
            nxt = 1 - b
            wait_gather(slot)

            @pl.when(c + 1 < CHUNKS_PER_WORKER)
            def _start_next():
                start_gather(c + 1, nxt)

            add_pos_and_emit(c, slot)
        return carry

    lax.fori_loop(0, CHUNKS_PER_WORKER // 2, outer, 0)


@jax.jit
def kernel(x, token_table, pos_table):
    x_flat = x.reshape(BATCH * SEQ).astype(jnp.int32)

    mesh = plsc.VectorSubcoreMesh(core_axis_name="c", subcore_axis_name="s")
    run = functools.partial(
        pl.kernel,
        mesh=mesh,
        compiler_params=pltpu.CompilerParams(use_tc_tiling_on_sc=False),
        out_type=jax.ShapeDtypeStruct((BATCH * SEQ, EMBED_DIM), jnp.float32),
        scratch_types=[
            pltpu.VMEM((ROWS_PER_WORKER,), jnp.int32),
            pltpu.VMEM((MAX_LEN, EMBED_DIM), jnp.float32),
            pltpu.VMEM((CHUNK, EMBED_DIM), jnp.float32),
            pltpu.VMEM((CHUNK, EMBED_DIM), jnp.float32),
            pltpu.SemaphoreType.DMA,
            pltpu.SemaphoreType.DMA,
        ],
    )(_sc_kernel_body)
    out = run(x_flat, token_table, pos_table)
    return out.reshape(BATCH, SEQ, EMBED_DIM)
